# speedup vs baseline: 8.5975x; 8.5975x over previous
import functools

import jax
import jax.numpy as jnp
from jax import lax
from jax.experimental import pallas as pl
from jax.experimental.pallas import tpu as pltpu
from jax.experimental.pallas import tpu_sc as plsc

N, E, D, H, C, G = 10000, 320000, 128, 128, 40, 64
EPS = 1e-7

NC, NS, LANES = 2, 16, 16
CHUNK = 128
CHUNKS_PER_SUB = 157
EDGES_PER_SUB = CHUNKS_PER_SUB * CHUNK
E_PAD = EDGES_PER_SUB * NS
ACC_ROWS = 10240
ROWS_PER_SUB = ACC_ROWS // NS


def _edge_pass(w_tab, q_tab, src_pad, dst2d):
    mesh = plsc.VectorSubcoreMesh(
        core_axis_name="c", subcore_axis_name="s", num_cores=NC, num_subcores=NS
    )
    out_ty = (
        jax.ShapeDtypeStruct((ACC_ROWS, D), jnp.float32),
        jax.ShapeDtypeStruct((ACC_ROWS, D), jnp.float32),
    )

    @functools.partial(
        pl.kernel,
        out_type=out_ty,
        mesh=mesh,
        scratch_types=[
            pltpu.VMEM((CHUNK,), jnp.int32),
            pltpu.VMEM((1, CHUNK), jnp.int32),
            pltpu.VMEM((CHUNK, D), jnp.float32),
            pltpu.VMEM_SHARED((ACC_ROWS, D), jnp.float32),
            pltpu.SemaphoreType.DMA,
        ],
    )
    def k(w_hbm, q_hbm, src_hbm, dst_hbm, den_hbm, num_hbm,
          src_v, dst_v, rows_v, acc, sem):
        c = lax.axis_index("c")
        s = lax.axis_index("s")

        @pl.loop(0, CHUNK)
        def _(r):
            @pl.loop(0, D // LANES)
            def _(j):
                rows_v[r, pl.ds(j * LANES, LANES)] = jnp.zeros((LANES,), jnp.float32)

        @pl.loop(0, ROWS_PER_SUB // CHUNK)
        def _(b):
            pltpu.sync_copy(
                rows_v, acc.at[pl.ds(s * ROWS_PER_SUB + b * CHUNK, CHUNK)]
            )

        plsc.subcore_barrier()

        base = s * EDGES_PER_SUB

        def chunk_body(g, tab_hbm):
            off = base + g * CHUNK
            pltpu.sync_copy(src_hbm.at[pl.ds(off, CHUNK)], src_v)
            row = (base // CHUNK) + g
            pltpu.sync_copy(dst_hbm.at[pl.ds(row, 1)], dst_v)
            pltpu.async_copy(tab_hbm.at[src_v], rows_v, sem).wait()
            pltpu.sync_copy(rows_v, acc.at[dst_v.at[0]], add=True)

        @pl.when(c == 0)
        def _():
            @pl.loop(0, CHUNKS_PER_SUB)
            def _(g):
                chunk_body(g, w_hbm)

        @pl.when(c == 1)
        def _():
            @pl.loop(0, CHUNKS_PER_SUB)
            def _(g):
                chunk_body(g, q_hbm)

        plsc.subcore_barrier()

        sl = pl.ds(s * ROWS_PER_SUB, ROWS_PER_SUB)

        @pl.when(c == 0)
        def _():
            pltpu.sync_copy(acc.at[sl], den_hbm.at[sl])

        @pl.when(c == 1)
        def _():
            pltpu.sync_copy(acc.at[sl], num_hbm.at[sl])

    return k(w_tab, q_tab, src_pad, dst2d)


def _tables_body(x_ref, w_ref, q_ref):
    p = jnp.maximum(x_ref[...], 0.0) + EPS
    c = jnp.max(p, axis=0, keepdims=True)
    w = jnp.exp(p - c)
    w_ref[...] = w
    q_ref[...] = p * w


def _mlp(y, W1, b1, g1, be1, W2, b2):
    h = jnp.dot(y, W1, preferred_element_type=jnp.float32) + b1
    mu = jnp.mean(h, axis=0, keepdims=True)
    var = jnp.mean((h - mu) * (h - mu), axis=0, keepdims=True)
    h = (h - mu) * lax.rsqrt(var + 1e-5) * g1 + be1
    h = jnp.maximum(h, 0.0)
    return jnp.dot(h, W2, preferred_element_type=jnp.float32) + b2


def _layer_body(den_ref, num_ref, x_ref, W1_ref, b1_ref, g1_ref, be1_ref,
                W2_ref, b2_ref, h_ref, w_ref, q_ref):
    aggr = num_ref[:N, :] / (den_ref[:N, :] + 1e-16)
    y = aggr + x_ref[...]
    h = _mlp(y, W1_ref[...], b1_ref[...], g1_ref[...], be1_ref[...],
             W2_ref[...], b2_ref[...])
    h = jnp.maximum(h, 0.0)
    h_ref[...] = h
    p = h + EPS
    c = jnp.max(p, axis=0, keepdims=True)
    w = jnp.exp(p - c)
    w_ref[...] = w
    q_ref[...] = p * w


def _final_body(den_ref, num_ref, h_ref, W1_ref, b1_ref, g1_ref, be1_ref,
                W2_ref, b2_ref, batch_ref, fcW_ref, fcb_ref, out_ref):
    aggr = num_ref[:N, :] / (den_ref[:N, :] + 1e-16)
    y = aggr + h_ref[...]
    z = _mlp(y, W1_ref[...], b1_ref[...], g1_ref[...], be1_ref[...],
             W2_ref[...], b2_ref[...])
    z = jnp.maximum(z, 0.0)
    gids = lax.broadcasted_iota(jnp.int32, (G, N), 0)
    onehot = (gids == batch_ref[...]).astype(jnp.float32)
    sums = jnp.dot(onehot, z, preferred_element_type=jnp.float32)
    cnt = jnp.sum(onehot, axis=1, keepdims=True)
    pooled = sums / jnp.maximum(cnt, 1.0)
    out_ref[...] = (
        jnp.dot(pooled, fcW_ref[...], preferred_element_type=jnp.float32)
        + fcb_ref[...]
    )


def _f32(shape):
    return jax.ShapeDtypeStruct(shape, jnp.float32)


def kernel(x, edge_index, batch, c1_W1, c1_b1, c1_g1, c1_be1, c1_W2, c1_b2,
           c2_W1, c2_b1, c2_g1, c2_be1, c2_W2, c2_b2, fc_W, fc_b):
    src = edge_index[0]
    dst = edge_index[1]
    pad = E_PAD - E
    src_pad = jnp.concatenate([src, jnp.zeros((pad,), jnp.int32)])
    dst_pad = jnp.concatenate([dst, jnp.full((pad,), N, jnp.int32)])
    dst2d = dst_pad.reshape(E_PAD // CHUNK, CHUNK)
    batch2d = batch.reshape(1, N)

    w1t, q1t = pl.pallas_call(
        _tables_body, out_shape=(_f32((N, D)), _f32((N, D)))
    )(x)

    den1, num1 = _edge_pass(w1t, q1t, src_pad, dst2d)

    h, w2t, q2t = pl.pallas_call(
        _layer_body, out_shape=(_f32((N, H)), _f32((N, H)), _f32((N, H)))
    )(den1, num1, x, c1_W1, c1_b1, c1_g1, c1_be1, c1_W2, c1_b2)

    den2, num2 = _edge_pass(w2t, q2t, src_pad, dst2d)

    out = pl.pallas_call(_final_body, out_shape=_f32((G, C)))(
        den2, num2, h, c2_W1, c2_b1, c2_g1, c2_be1, c2_W2, c2_b2,
        batch2d, fc_W, fc_b)
    return out
